# baseline (speedup 1.0000x reference)
import functools

import jax
import jax.numpy as jnp
from jax import lax
from jax.experimental import pallas as pl
from jax.experimental.pallas import tpu as pltpu
from jax.experimental.pallas import tpu_sc as plsc

D_MODEL = 2048
HALF = D_MODEL // 2
LANES = 16
NUM_WORKERS = 32
K = 8
N_A = 6400

_MESH = plsc.VectorSubcoreMesh(core_axis_name="c", subcore_axis_name="s")
_PARAMS = pltpu.CompilerParams(needs_layout_passes=False)


def _pack_table(table):
    half = table.shape[1] // 2
    bf = table.astype(jnp.bfloat16)
    lo = lax.bitcast_convert_type(bf[:, :half], jnp.uint16).astype(jnp.uint32)
    hi = lax.bitcast_convert_type(bf[:, half:], jnp.uint16).astype(jnp.uint32)
    return lax.bitcast_convert_type(lo | (hi << 16), jnp.int32)


def _seq_rows_splat(sidx, off, i):
    rows16 = sidx[pl.ds(off, LANES)]
    return jnp.take_along_axis(rows16, jnp.full((LANES,), i, jnp.int32), axis=0)


def _phase_a(start, n_tok):
    per_w = n_tok // NUM_WORKERS
    n_chunks = per_w // K
    n_loop = n_chunks - 1
    NO = 3
    n_outer = n_loop // NO
    assert n_loop % NO == 0

    @functools.partial(
        pl.kernel, mesh=_MESH, compiler_params=_PARAMS, out_type=(),
        scratch_types=[
            pltpu.VMEM((per_w + LANES,), jnp.int32),
            pltpu.VMEM((per_w,), jnp.int32),
            pltpu.VMEM((64, HALF), jnp.int32),
            pltpu.VMEM((NO, K, D_MODEL), jnp.float32),
            pltpu.SemaphoreType.DMA,
            pltpu.SemaphoreType.DMA,
            pltpu.SemaphoreType.DMA,
            pltpu.SemaphoreType.DMA,
            pltpu.SemaphoreType.DMA,
            pltpu.SemaphoreType.DMA,
        ],
    )
    def k(seq_tok_hbm, struct_tok_hbm, seq_pk_hbm, struct_tab_hbm, out_hbm,
          sidx, tidx, seq_pk, buf_o,
          sem_o0, sem_o1, sem_o2, sem_w0, sem_w1, sem_w2):
        sem_o = (sem_o0, sem_o1, sem_o2)
        sem_w = (sem_w0, sem_w1, sem_w2)
        wid = lax.axis_index("s") * 2 + lax.axis_index("c")
        base = start + wid * per_w
        pltpu.sync_copy(seq_tok_hbm.at[pl.ds(base, per_w)],
                        sidx.at[pl.ds(0, per_w)])
        pltpu.sync_copy(struct_tok_hbm.at[pl.ds(base, per_w)], tidx)
        pltpu.sync_copy(seq_pk_hbm, seq_pk)

        def gather(off, bo):
            pltpu.async_copy(
                struct_tab_hbm.at[tidx.at[pl.ds(off, K)]], buf_o.at[bo],
                sem_o[bo])

        def wait_gather(off, bo):
            pltpu.make_async_copy(
                struct_tab_hbm.at[tidx.at[pl.ds(off, K)]], buf_o.at[bo],
                sem_o[bo]).wait()

        def writeback(off, bo):
            return pltpu.make_async_copy(
                buf_o.at[bo], out_hbm.at[pl.ds(base + off, K)], sem_w[bo])

        lanes_iota = lax.iota(jnp.int32, LANES)
        himask = jnp.full((LANES,), -65536, jnp.int32)

        def do_chunk(g_off, b):
            for i in range(K):
                splat = _seq_rows_splat(sidx, g_off, i)

                @plsc.parallel_loop(0, HALF, LANES, unroll=4)
                def _(j, b=b, i=i, splat=splat):
                    w = plsc.load_gather(seq_pk, [splat, lanes_iota + j])
                    lo = plsc.bitcast(lax.shift_left(w, 16), jnp.float32)
                    hi = plsc.bitcast(w & himask, jnp.float32)
                    plsc.addupdate(buf_o.at[b, i, pl.ds(j, LANES)], lo)
                    plsc.addupdate(buf_o.at[b, i, pl.ds(j + HALF, LANES)], hi)

        gather(0, 0)
        gather(K, 1)

        def outer(o, _):
            for b in range(NO):
                off = (o * NO + b) * K
                wait_gather(off, b)
                do_chunk(off, b)
                writeback(off, b).start()
                bo2 = (b + 2) % NO
                if b == 0:
                    @pl.when(o > 0)
                    def _():
                        writeback(off - K, bo2).wait()
                    gather(off + 2 * K, bo2)
                elif b == 1:
                    writeback(off - K, bo2).wait()
                    gather(off + 2 * K, bo2)
                else:
                    @pl.when(o < n_outer - 1)
                    def _():
                        writeback(off - K, bo2).wait()
                        gather(off + 2 * K, bo2)
            return 0

        lax.fori_loop(0, n_outer, outer, 0)

        last = n_loop * K
        wait_gather(last, 0)
        do_chunk(last, 0)
        writeback(last, 0).start()
        writeback(last - 2 * K, 1).wait()
        writeback(last - K, 2).wait()
        writeback(last, 0).wait()

    return k


def _phase_b(start, n_tok):
    per_w = n_tok // NUM_WORKERS
    n_chunks = per_w // K
    n_loop = n_chunks - 1
    NB = 2
    n_outer = n_loop // NB
    assert n_loop % NB == 0

    @functools.partial(
        pl.kernel, mesh=_MESH, compiler_params=_PARAMS, out_type=(),
        scratch_types=[
            pltpu.VMEM((per_w + LANES,), jnp.int32),
            pltpu.VMEM((per_w,), jnp.int32),
            pltpu.VMEM((64, HALF), jnp.int32),
            pltpu.VMEM((NB, K, HALF), jnp.int32),
            pltpu.VMEM((NB, K, D_MODEL), jnp.float32),
            pltpu.SemaphoreType.DMA,
            pltpu.SemaphoreType.DMA,
            pltpu.SemaphoreType.DMA,
            pltpu.SemaphoreType.DMA,
        ],
    )
    def k(seq_tok_hbm, struct_tok_hbm, seq_pk_hbm, struct_pk_hbm, out_hbm,
          sidx, tidx, seq_pk, buf_g, buf_w,
          sem_g0, sem_g1, sem_w0, sem_w1):
        sem_g = (sem_g0, sem_g1)
        sem_w = (sem_w0, sem_w1)
        wid = lax.axis_index("s") * 2 + lax.axis_index("c")
        base = start + wid * per_w
        pltpu.sync_copy(seq_tok_hbm.at[pl.ds(base, per_w)],
                        sidx.at[pl.ds(0, per_w)])
        pltpu.sync_copy(struct_tok_hbm.at[pl.ds(base, per_w)], tidx)
        pltpu.sync_copy(seq_pk_hbm, seq_pk)

        def gather(off, b):
            pltpu.async_copy(
                struct_pk_hbm.at[tidx.at[pl.ds(off, K)]], buf_g.at[b],
                sem_g[b])

        def wait_gather(off, b):
            pltpu.make_async_copy(
                struct_pk_hbm.at[tidx.at[pl.ds(off, K)]], buf_g.at[b],
                sem_g[b]).wait()

        def writeback(off, b):
            return pltpu.make_async_copy(
                buf_w.at[b], out_hbm.at[pl.ds(base + off, K)], sem_w[b])

        lanes_iota = lax.iota(jnp.int32, LANES)
        himask = jnp.full((LANES,), -65536, jnp.int32)

        def do_chunk(g_off, b):
            for i in range(K):
                splat = _seq_rows_splat(sidx, g_off, i)

                @plsc.parallel_loop(0, HALF, LANES, unroll=4)
                def _(j, b=b, i=i, splat=splat):
                    wq = plsc.load_gather(seq_pk, [splat, lanes_iota + j])
                    ws = buf_g[b, i, pl.ds(j, LANES)]
                    lo = (plsc.bitcast(lax.shift_left(wq, 16), jnp.float32)
                          + plsc.bitcast(lax.shift_left(ws, 16), jnp.float32))
                    hi = (plsc.bitcast(wq & himask, jnp.float32)
                          + plsc.bitcast(ws & himask, jnp.float32))
                    buf_w[b, i, pl.ds(j, LANES)] = lo
                    buf_w[b, i, pl.ds(j + HALF, LANES)] = hi

        gather(0, 0)
        gather(K, 1)

        def outer(o, _):
            for b in range(NB):
                off = (o * NB + b) * K
                wait_gather(off, b)
                @pl.when(o > 0)
                def _():
                    writeback(off - NB * K, b).wait()
                do_chunk(off, b)
                writeback(off, b).start()
                if b == 0:
                    gather(off + NB * K, b)
                else:
                    @pl.when(o < n_outer - 1)
                    def _():
                        gather(off + NB * K, b)
            return 0

        lax.fori_loop(0, n_outer, outer, 0)

        last = n_loop * K
        wait_gather(last, 0)
        writeback(last - 2 * K, 0).wait()
        do_chunk(last, 0)
        writeback(last, 0).start()
        writeback(last - K, 1).wait()
        writeback(last, 0).wait()

    return k


@jax.jit
def _gather_add(seq_tok, struct_tok, seq_table, struct_table):
    n = seq_tok.shape[0]
    seq_pk = _pack_table(seq_table)
    struct_pk = _pack_table(struct_table)
    out_ref = jax.new_ref(lax.empty((n, D_MODEL), jnp.float32))
    _phase_a(0, N_A)(seq_tok, struct_tok, seq_pk, struct_table, out_ref)
    _phase_b(N_A, n - N_A)(seq_tok, struct_tok, seq_pk, struct_pk, out_ref)
    return out_ref[...]


def kernel(sequence_tokens, structure_tokens, seq_table, struct_table):
    b, s = sequence_tokens.shape
    n = b * s
    seq_tok = sequence_tokens.reshape(n).astype(jnp.int32)
    struct_tok = structure_tokens.reshape(n).astype(jnp.int32)
    out = _gather_add(seq_tok, struct_tok, seq_table, struct_table)
    return out.reshape(b, s, D_MODEL)


# speedup vs baseline: 1.1591x; 1.1591x over previous
import functools

import jax
import jax.numpy as jnp
from jax import lax
from jax.experimental import pallas as pl
from jax.experimental.pallas import tpu as pltpu
from jax.experimental.pallas import tpu_sc as plsc

D_MODEL = 2048
LANES = 16
NUM_WORKERS = 32
K = 8
NO = 3


def _pack_seq_table(seq_table):
    bf = seq_table.astype(jnp.bfloat16).reshape(seq_table.shape[0], -1, 2, LANES)
    bits = lax.bitcast_convert_type(bf, jnp.uint16).astype(jnp.uint32)
    words = bits[:, :, 0, :] | (bits[:, :, 1, :] << 16)
    return lax.bitcast_convert_type(words, jnp.int32).reshape(
        seq_table.shape[0], seq_table.shape[1] // 2)


@jax.jit
def _gather_add(seq_tok, struct_tok, seq_packed, struct_table):
    n = seq_tok.shape[0]
    v_seq = seq_packed.shape[0]
    per_w = n // NUM_WORKERS
    n_chunks = per_w // K
    n_loop = n_chunks - 1
    n_outer = n_loop // NO
    mesh = plsc.VectorSubcoreMesh(core_axis_name="c", subcore_axis_name="s")

    @functools.partial(
        pl.kernel,
        mesh=mesh,
        compiler_params=pltpu.CompilerParams(needs_layout_passes=False),
        out_type=jax.ShapeDtypeStruct((n, D_MODEL), jnp.float32),
        scratch_types=[
            pltpu.VMEM((per_w + LANES,), jnp.int32),
            pltpu.VMEM((per_w,), jnp.int32),
            pltpu.VMEM((v_seq, D_MODEL // 2), jnp.int32),
            pltpu.VMEM((NO, K, D_MODEL), jnp.float32),
            pltpu.SemaphoreType.DMA,
            pltpu.SemaphoreType.DMA,
            pltpu.SemaphoreType.DMA,
            pltpu.SemaphoreType.DMA,
            pltpu.SemaphoreType.DMA,
            pltpu.SemaphoreType.DMA,
        ],
    )
    def k(seq_tok_hbm, struct_tok_hbm, seq_pk_hbm, struct_tab_hbm, out_hbm,
          sidx, tidx, seq_pk, buf_o,
          sem_o0, sem_o1, sem_o2, sem_w0, sem_w1, sem_w2):
        sem_o = (sem_o0, sem_o1, sem_o2)
        sem_w = (sem_w0, sem_w1, sem_w2)
        wid = lax.axis_index("s") * 2 + lax.axis_index("c")
        base = wid * per_w
        pltpu.sync_copy(seq_tok_hbm.at[pl.ds(base, per_w)],
                        sidx.at[pl.ds(0, per_w)])
        pltpu.sync_copy(struct_tok_hbm.at[pl.ds(base, per_w)], tidx)
        pltpu.sync_copy(seq_pk_hbm, seq_pk)

        def gather(off, bo):
            pltpu.async_copy(
                struct_tab_hbm.at[tidx.at[pl.ds(off, K)]], buf_o.at[bo],
                sem_o[bo])

        def wait_gather(off, bo):
            pltpu.make_async_copy(
                struct_tab_hbm.at[tidx.at[pl.ds(off, K)]], buf_o.at[bo],
                sem_o[bo]).wait()

        def writeback(off, bo):
            return pltpu.make_async_copy(
                buf_o.at[bo], out_hbm.at[pl.ds(base + off, K)], sem_w[bo])

        lanes_iota = lax.iota(jnp.int32, LANES)
        himask = jnp.full((LANES,), -65536, jnp.int32)

        def do_chunk(g_off, b):
            rows16 = sidx[pl.ds(g_off, LANES)]
            for i in range(K):
                splat = jnp.take_along_axis(
                    rows16, jnp.full((LANES,), i, jnp.int32), axis=0)

                @plsc.parallel_loop(0, D_MODEL, 2 * LANES, unroll=4)
                def _(j, b=b, i=i, splat=splat):
                    w = plsc.load_gather(
                        seq_pk, [splat, lanes_iota + lax.shift_right_logical(j, 1)])
                    lo = plsc.bitcast(lax.shift_left(w, 16), jnp.float32)
                    hi = plsc.bitcast(w & himask, jnp.float32)
                    plsc.addupdate(buf_o.at[b, i, pl.ds(j, LANES)], lo)
                    plsc.addupdate(buf_o.at[b, i, pl.ds(j + LANES, LANES)], hi)

        gather(0, 0)
        gather(K, 1)

        def outer(o, _):
            for b in range(NO):
                off = (o * NO + b) * K
                wait_gather(off, b)
                bo2 = (b + 2) % NO
                if b == 0:
                    @pl.when(o > 0)
                    def _():
                        writeback(off - K, bo2).wait()
                    gather(off + 2 * K, bo2)
                elif b == 1:
                    writeback(off - K, bo2).wait()
                    gather(off + 2 * K, bo2)
                else:
                    @pl.when(o < n_outer - 1)
                    def _():
                        writeback(off - K, bo2).wait()
                        gather(off + 2 * K, bo2)
                do_chunk(off, b)
                writeback(off, b).start()
            return 0

        lax.fori_loop(0, n_outer, outer, 0)

        last = n_loop * K
        wait_gather(last, 0)
        do_chunk(last, 0)
        writeback(last, 0).start()
        writeback(last - 2 * K, 1).wait()
        writeback(last - K, 2).wait()
        writeback(last, 0).wait()

    return k(seq_tok, struct_tok, seq_packed, struct_table)


def kernel(sequence_tokens, structure_tokens, seq_table, struct_table):
    b, s = sequence_tokens.shape
    n = b * s
    seq_tok = sequence_tokens.reshape(n).astype(jnp.int32)
    struct_tok = structure_tokens.reshape(n).astype(jnp.int32)
    out = _gather_add(seq_tok, struct_tok, _pack_seq_table(seq_table),
                      struct_table)
    return out.reshape(b, s, D_MODEL)
